# baseline (speedup 1.0000x reference)
import functools
import math

import jax
import jax.numpy as jnp
from jax import lax
from jax.experimental import pallas as pl
from jax.experimental.pallas import tpu as pltpu
from jax.experimental.pallas import tpu_sc as plsc

D = 16
NB = 10
RN = 100
MAX_R = 3.5
S_MAX = MAX_R * MAX_R
KTAB = 8192
SILU_NORM = 1.6790590095608847
EPS = 1e-5
SIN8 = math.sin(math.pi / 8.0)
COS8 = math.cos(math.pi / 8.0)

NC = 2
NS = 16
NW = NC * NS
CHUNK = 1024


def _silu(v):
    return v * (1.0 / (1.0 + jnp.exp(-v)))


def _edge_s_kernel(ev_ref, s_ref):
    v = ev_ref[...]
    s_ref[...] = jnp.sum(v * v, axis=1)


def _edge_s(edge_vec, E):
    BE = 5120
    assert E % BE == 0
    return pl.pallas_call(
        _edge_s_kernel,
        grid=(E // BE,),
        in_specs=[pl.BlockSpec((BE, 3), lambda i: (i, 0))],
        out_specs=pl.BlockSpec((BE,), lambda i: (i,)),
        out_shape=jax.ShapeDtypeStruct((E,), jnp.float32),
    )(edge_vec)


def _table_kernel(wf1_ref, wf2_ref, tab_ref):
    k = lax.broadcasted_iota(jnp.int32, (KTAB, 1), 0).astype(jnp.float32)
    s = k * (S_MAX / (KTAB - 1))
    r = jnp.sqrt(s)
    centers = (lax.broadcasted_iota(jnp.int32, (KTAB, NB), 1).astype(jnp.float32)
               * (MAX_R / (NB - 1)))
    step = MAX_R / (NB - 1)
    diff = (r - centers) / step
    emb = jnp.exp(-(diff * diff)) * ((NB ** 0.5) / 1.12)
    u = 2.0 * (r / MAX_R - 1.0)
    y = (1.0 - jnp.cos(math.pi * u)) * 0.5
    y = jnp.where(u > 0.0, 0.0, y)
    y = jnp.where(u < -1.0, 1.0, y)
    h = _silu(jnp.dot(emb, wf1_ref[0], preferred_element_type=jnp.float32)
              * (1.0 / (NB ** 0.5))) * SILU_NORM
    w = jnp.dot(h, wf2_ref[0], preferred_element_type=jnp.float32) * (1.0 / (RN ** 0.5))
    G = w * y * (1.0 / (32.0 ** 0.5))
    dG = jnp.concatenate([G[1:] - G[:-1], jnp.zeros((1, D), jnp.float32)], axis=0)
    tab_ref[0] = jnp.concatenate([G, dG], axis=1)


def _build_tables(Wf1s, Wf2s):
    return pl.pallas_call(
        _table_kernel,
        grid=(3,),
        in_specs=[
            pl.BlockSpec((1, NB, RN), lambda l: (l, 0, 0)),
            pl.BlockSpec((1, RN, D), lambda l: (l, 0, 0)),
        ],
        out_specs=pl.BlockSpec((1, KTAB, 2 * D), lambda l: (l, 0, 0)),
        out_shape=jax.ShapeDtypeStruct((3, KTAB, 2 * D), jnp.float32),
    )(Wf1s, Wf2s)


def _pre_kernel(h_ref, wl1_ref, wsc_ref, x1_ref, sc_ref):
    h = h_ref[...]
    x1_ref[...] = jnp.dot(h, wl1_ref[...], preferred_element_type=jnp.float32) * 0.25
    sc_ref[...] = jnp.dot(h, wsc_ref[...], preferred_element_type=jnp.float32) * (SIN8 * 0.25)


def _pre(h, Wl1, Wsc, N, BN):
    return pl.pallas_call(
        _pre_kernel,
        grid=(N // BN,),
        in_specs=[
            pl.BlockSpec((BN, D), lambda i: (i, 0)),
            pl.BlockSpec((D, D), lambda i: (0, 0)),
            pl.BlockSpec((D, D), lambda i: (0, 0)),
        ],
        out_specs=[
            pl.BlockSpec((BN, D), lambda i: (i, 0)),
            pl.BlockSpec((BN, D), lambda i: (i, 0)),
        ],
        out_shape=[
            jax.ShapeDtypeStruct((N, D), jnp.float32),
            jax.ShapeDtypeStruct((N, D), jnp.float32),
        ],
    )(h, Wl1, Wsc)


def _q1_kernel(aggp_ref, sc_ref, wl2_ref, g_ref, sums_ref):
    c = sc_ref[...] + jnp.dot(aggp_ref[...], wl2_ref[...],
                              preferred_element_type=jnp.float32) * (COS8 * 0.25)
    g = _silu(c) * SILU_NORM
    g_ref[...] = g
    sg = jnp.sum(g, axis=0)
    sg2 = jnp.sum(g * g, axis=0)
    part = jnp.concatenate(
        [sg[None], sg2[None], jnp.zeros((6, D), jnp.float32)], axis=0)

    @pl.when(pl.program_id(0) == 0)
    def _():
        sums_ref[...] = jnp.zeros_like(sums_ref)

    sums_ref[...] += part


def _q1(aggp, sc, Wl2, N, BN):
    return pl.pallas_call(
        _q1_kernel,
        grid=(N // BN,),
        in_specs=[
            pl.BlockSpec((BN, D), lambda i: (i, 0)),
            pl.BlockSpec((BN, D), lambda i: (i, 0)),
            pl.BlockSpec((D, D), lambda i: (0, 0)),
        ],
        out_specs=[
            pl.BlockSpec((BN, D), lambda i: (i, 0)),
            pl.BlockSpec((8, D), lambda i: (0, 0)),
        ],
        out_shape=[
            jax.ShapeDtypeStruct((N, D), jnp.float32),
            jax.ShapeDtypeStruct((8, D), jnp.float32),
        ],
    )(aggp, sc, Wl2)


def _q2p_kernel(g_ref, hin_ref, sums_ref, bnw_ref, bnb_ref, wl1_ref, wsc_ref,
                h_ref, x1_ref, sc_ref, *, N):
    inv_n = 1.0 / N
    mean = sums_ref[0:1, :] * inv_n
    var = sums_ref[1:2, :] * inv_n - mean * mean
    rstd = lax.rsqrt(var + EPS)
    gn = (g_ref[...] - mean) * rstd * bnw_ref[...] + bnb_ref[...]
    h = gn + hin_ref[...]
    h_ref[...] = h
    x1_ref[...] = jnp.dot(h, wl1_ref[...], preferred_element_type=jnp.float32) * 0.25
    sc_ref[...] = jnp.dot(h, wsc_ref[...], preferred_element_type=jnp.float32) * (SIN8 * 0.25)


def _q2p(g, hin, sums, bnw, bnb, Wl1, Wsc, N, BN):
    return pl.pallas_call(
        functools.partial(_q2p_kernel, N=N),
        grid=(N // BN,),
        in_specs=[
            pl.BlockSpec((BN, D), lambda i: (i, 0)),
            pl.BlockSpec((BN, D), lambda i: (i, 0)),
            pl.BlockSpec((8, D), lambda i: (0, 0)),
            pl.BlockSpec((1, D), lambda i: (0, 0)),
            pl.BlockSpec((1, D), lambda i: (0, 0)),
            pl.BlockSpec((D, D), lambda i: (0, 0)),
            pl.BlockSpec((D, D), lambda i: (0, 0)),
        ],
        out_specs=[
            pl.BlockSpec((BN, D), lambda i: (i, 0)),
            pl.BlockSpec((BN, D), lambda i: (i, 0)),
            pl.BlockSpec((BN, D), lambda i: (i, 0)),
        ],
        out_shape=[
            jax.ShapeDtypeStruct((N, D), jnp.float32),
            jax.ShapeDtypeStruct((N, D), jnp.float32),
            jax.ShapeDtypeStruct((N, D), jnp.float32),
        ],
    )(g, hin, sums, bnw, bnb, Wl1, Wsc)


def _final_kernel(aggp_ref, sc_ref, wl2_ref, osum_ref, *, N):
    c = sc_ref[...] + jnp.dot(aggp_ref[...], wl2_ref[...],
                              preferred_element_type=jnp.float32) * (COS8 * 0.25)
    part = jnp.sum(c, axis=0) * (1.0 / (N ** 0.5))
    block = jnp.concatenate([part[None], jnp.zeros((7, D), jnp.float32)], axis=0)

    @pl.when(pl.program_id(0) == 0)
    def _():
        osum_ref[...] = jnp.zeros_like(osum_ref)

    osum_ref[...] += block


def _final(aggp, sc, Wl2, N, BN):
    return pl.pallas_call(
        functools.partial(_final_kernel, N=N),
        grid=(N // BN,),
        in_specs=[
            pl.BlockSpec((BN, D), lambda i: (i, 0)),
            pl.BlockSpec((BN, D), lambda i: (i, 0)),
            pl.BlockSpec((D, D), lambda i: (0, 0)),
        ],
        out_specs=pl.BlockSpec((8, D), lambda i: (0, 0)),
        out_shape=jax.ShapeDtypeStruct((8, D), jnp.float32),
    )(aggp, sc, Wl2)


def _sc_edge_call(N, EP):
    nchunks = EP // CHUNK
    NH = N // 2
    rpt8 = (NH // NS) & ~7
    tail_off = rpt8 * NS
    tail = NH - tail_off
    mesh = plsc.VectorSubcoreMesh(core_axis_name="c", subcore_axis_name="s")
    scale = (KTAB - 1) / S_MAX

    def body(x1_hbm, src_hbm, dst_hbm, s_hbm, tab_hbm, zero_hbm, out_hbm,
             src_v, dst_v, s_v, idx_v, frac_v, tab_v, xg_v, ef_v,
             agg_sh, sem_a, sem_b):
        c = lax.axis_index("c")
        t = lax.axis_index("s")
        node_lo = c * NH
        pltpu.sync_copy(zero_hbm.at[pl.ds(t * rpt8, rpt8)],
                        agg_sh.at[pl.ds(t * rpt8, rpt8)])
        if tail:
            @pl.when(t == NS - 1)
            def _():
                pltpu.sync_copy(zero_hbm.at[pl.ds(tail_off, tail)],
                                agg_sh.at[pl.ds(tail_off, tail)])
        plsc.subcore_barrier()

        def chunk_body(ci, carry):
            b0 = t * EP + ci * CHUNK
            pltpu.sync_copy(src_hbm.at[pl.ds(b0, CHUNK)], src_v)
            pltpu.sync_copy(dst_hbm.at[pl.ds(b0, CHUNK)], dst_v)
            pltpu.sync_copy(s_hbm.at[pl.ds(b0, CHUNK)], s_v)

            def lane_body(k, carry2):
                sl = s_v[pl.ds(k * 16, 16)]
                p = jnp.minimum(sl * scale, KTAB - 1.0)
                i = jnp.minimum(p.astype(jnp.int32), KTAB - 2)
                loc = dst_v[pl.ds(k * 16, 16)] - node_lo
                ok = (loc >= 0) & (loc < NH)
                idx_v[pl.ds(k * 16, 16)] = jnp.where(ok, i, KTAB - 1)
                dst_v[pl.ds(k * 16, 16)] = jnp.where(ok, loc, 0)
                frac_v[pl.ds(k * 16, 16)] = p - i.astype(jnp.float32)
                return carry2

            lax.fori_loop(0, CHUNK // 16, lane_body, 0, unroll=4)
            cp_a = pltpu.async_copy(tab_hbm.at[idx_v], tab_v, sem_a)
            cp_b = pltpu.async_copy(x1_hbm.at[src_v], xg_v, sem_b)
            cp_a.wait()
            cp_b.wait()

            def row_body(k, carry3):
                f16 = frac_v[pl.ds(k * 16, 16)]
                for j in range(16):
                    e = k * 16 + j
                    g = tab_v[e, 0:D] + f16[j] * tab_v[e, D:2 * D]
                    ef_v[e, :] = g * xg_v[e, :]
                return carry3

            lax.fori_loop(0, CHUNK // 16, row_body, 0, unroll=2)
            pltpu.sync_copy(ef_v, agg_sh.at[dst_v], add=True)
            return carry

        lax.fori_loop(0, nchunks, chunk_body, 0)
        plsc.subcore_barrier()
        pltpu.sync_copy(agg_sh.at[pl.ds(t * rpt8, rpt8)],
                        out_hbm.at[pl.ds(node_lo + t * rpt8, rpt8)])
        if tail:
            @pl.when(t == NS - 1)
            def _():
                pltpu.sync_copy(agg_sh.at[pl.ds(tail_off, tail)],
                                out_hbm.at[pl.ds(node_lo + tail_off, tail)])

    return pl.kernel(
        body,
        out_type=jax.ShapeDtypeStruct((N, D), jnp.float32),
        mesh=mesh,
        compiler_params=pltpu.CompilerParams(use_tc_tiling_on_sc=False),
        scratch_types=[
            pltpu.VMEM((CHUNK,), jnp.int32),
            pltpu.VMEM((CHUNK,), jnp.int32),
            pltpu.VMEM((CHUNK,), jnp.float32),
            pltpu.VMEM((CHUNK,), jnp.int32),
            pltpu.VMEM((CHUNK,), jnp.float32),
            pltpu.VMEM((CHUNK, 2 * D), jnp.float32),
            pltpu.VMEM((CHUNK, D), jnp.float32),
            pltpu.VMEM((CHUNK, D), jnp.float32),
            pltpu.VMEM_SHARED((N // 2, D), jnp.float32),
            pltpu.SemaphoreType.DMA,
            pltpu.SemaphoreType.DMA,
        ],
    )


def kernel(x, pos, edge_index, edge_vec,
           W0_sc, W0_lin1, W0_fc1, W0_fc2, W0_lin2,
           W1_sc, W1_lin1, W1_fc1, W1_fc2, W1_lin2,
           W2_sc, W2_lin1, W2_fc1, W2_fc2, W2_lin2,
           bn0_w, bn0_b, bn1_w, bn1_b):
    N = x.shape[0]
    E = edge_index.shape[1]
    assert N % NS == 0
    BN = 5000
    assert N % BN == 0

    s = _edge_s(edge_vec, E)
    epad = -E % (NS * CHUNK)
    EPAD = E + epad
    src = edge_index[0]
    dst = edge_index[1]
    if epad:
        s = jnp.concatenate([s, jnp.full((epad,), 2.0 * S_MAX, jnp.float32)])
        src = jnp.concatenate([src, jnp.zeros((epad,), jnp.int32)])
        dst = jnp.concatenate([dst, jnp.zeros((epad,), jnp.int32)])
    EP = EPAD // NS

    tables = _build_tables(jnp.stack([W0_fc1, W1_fc1, W2_fc1]),
                           jnp.stack([W0_fc2, W1_fc2, W2_fc2]))
    zero_nd = jnp.zeros((N // 2, D), jnp.float32)
    sc_edge = _sc_edge_call(N, EP)

    x1, sc0 = _pre(x, W0_lin1, W0_sc, N, BN)
    aggp = sc_edge(x1, src, dst, s, tables[0], zero_nd)
    g0, sums0 = _q1(aggp, sc0, W0_lin2, N, BN)
    h1, x1, sc1 = _q2p(g0, x, sums0, bn0_w[None], bn0_b[None],
                       W1_lin1, W1_sc, N, BN)
    aggp = sc_edge(x1, src, dst, s, tables[1], zero_nd)
    g1, sums1 = _q1(aggp, sc1, W1_lin2, N, BN)
    _, x1, sc2 = _q2p(g1, h1, sums1, bn1_w[None], bn1_b[None],
                      W2_lin1, W2_sc, N, BN)
    aggp = sc_edge(x1, src, dst, s, tables[2], zero_nd)
    osum = _final(aggp, sc2, W2_lin2, N, BN)
    return osum[0:1, :]


# speedup vs baseline: 1.9004x; 1.9004x over previous
import functools
import math

import jax
import jax.numpy as jnp
from jax import lax
from jax.experimental import pallas as pl
from jax.experimental.pallas import tpu as pltpu
from jax.experimental.pallas import tpu_sc as plsc

D = 16
NB = 10
RN = 100
MAX_R = 3.5
S_MAX = MAX_R * MAX_R
KTAB = 65536
TBK = 8192
SILU_NORM = 1.6790590095608847
EPS = 1e-5
SIN8 = math.sin(math.pi / 8.0)
COS8 = math.cos(math.pi / 8.0)

NC = 2
NS = 16
NW = NC * NS
CHUNK = 1024


def _silu(v):
    return v * (1.0 / (1.0 + jnp.exp(-v)))


def _edge_s_kernel(ev_ref, s_ref):
    v = ev_ref[...]
    s_ref[...] = jnp.sum(v * v, axis=1)


def _edge_s(edge_vec, E):
    BE = 5120
    assert E % BE == 0
    return pl.pallas_call(
        _edge_s_kernel,
        grid=(E // BE,),
        in_specs=[pl.BlockSpec((BE, 3), lambda i: (i, 0))],
        out_specs=pl.BlockSpec((BE,), lambda i: (i,)),
        out_shape=jax.ShapeDtypeStruct((E,), jnp.float32),
    )(edge_vec)


def _table_kernel(wf1_ref, wf2_ref, tab_ref):
    k0 = pl.program_id(1) * TBK
    k = (lax.broadcasted_iota(jnp.int32, (TBK, 1), 0) + k0).astype(jnp.float32)
    s = k * (S_MAX / (KTAB - 1))
    r = jnp.sqrt(s)
    centers = (lax.broadcasted_iota(jnp.int32, (TBK, NB), 1).astype(jnp.float32)
               * (MAX_R / (NB - 1)))
    step = MAX_R / (NB - 1)
    diff = (r - centers) / step
    emb = jnp.exp(-(diff * diff)) * ((NB ** 0.5) / 1.12)
    u = 2.0 * (r / MAX_R - 1.0)
    y = (1.0 - jnp.cos(math.pi * u)) * 0.5
    y = jnp.where(u > 0.0, 0.0, y)
    y = jnp.where(u < -1.0, 1.0, y)
    h = _silu(jnp.dot(emb, wf1_ref[0], preferred_element_type=jnp.float32)
              * (1.0 / (NB ** 0.5))) * SILU_NORM
    w = jnp.dot(h, wf2_ref[0], preferred_element_type=jnp.float32) * (1.0 / (RN ** 0.5))
    tab_ref[0] = w * y * (1.0 / (32.0 ** 0.5))


def _build_tables(Wf1s, Wf2s):
    return pl.pallas_call(
        _table_kernel,
        grid=(3, KTAB // TBK),
        in_specs=[
            pl.BlockSpec((1, NB, RN), lambda l, k: (l, 0, 0)),
            pl.BlockSpec((1, RN, D), lambda l, k: (l, 0, 0)),
        ],
        out_specs=pl.BlockSpec((1, TBK, D), lambda l, k: (l, k, 0)),
        out_shape=jax.ShapeDtypeStruct((3, KTAB, D), jnp.float32),
    )(Wf1s, Wf2s)


def _pre_kernel(h_ref, wl1_ref, wsc_ref, x1_ref, sc_ref):
    h = h_ref[...]
    x1_ref[...] = jnp.dot(h, wl1_ref[...], preferred_element_type=jnp.float32) * 0.25
    sc_ref[...] = jnp.dot(h, wsc_ref[...], preferred_element_type=jnp.float32) * (SIN8 * 0.25)


def _pre(h, Wl1, Wsc, N, BN):
    return pl.pallas_call(
        _pre_kernel,
        grid=(N // BN,),
        in_specs=[
            pl.BlockSpec((BN, D), lambda i: (i, 0)),
            pl.BlockSpec((D, D), lambda i: (0, 0)),
            pl.BlockSpec((D, D), lambda i: (0, 0)),
        ],
        out_specs=[
            pl.BlockSpec((BN, D), lambda i: (i, 0)),
            pl.BlockSpec((BN, D), lambda i: (i, 0)),
        ],
        out_shape=[
            jax.ShapeDtypeStruct((N, D), jnp.float32),
            jax.ShapeDtypeStruct((N, D), jnp.float32),
        ],
    )(h, Wl1, Wsc)


def _q1_kernel(aggp_ref, sc_ref, wl2_ref, g_ref, sums_ref):
    c = sc_ref[...] + jnp.dot(aggp_ref[...], wl2_ref[...],
                              preferred_element_type=jnp.float32) * (COS8 * 0.25)
    g = _silu(c) * SILU_NORM
    g_ref[...] = g
    sg = jnp.sum(g, axis=0)
    sg2 = jnp.sum(g * g, axis=0)
    part = jnp.concatenate(
        [sg[None], sg2[None], jnp.zeros((6, D), jnp.float32)], axis=0)

    @pl.when(pl.program_id(0) == 0)
    def _():
        sums_ref[...] = jnp.zeros_like(sums_ref)

    sums_ref[...] += part


def _q1(aggp, sc, Wl2, N, BN):
    return pl.pallas_call(
        _q1_kernel,
        grid=(N // BN,),
        in_specs=[
            pl.BlockSpec((BN, D), lambda i: (i, 0)),
            pl.BlockSpec((BN, D), lambda i: (i, 0)),
            pl.BlockSpec((D, D), lambda i: (0, 0)),
        ],
        out_specs=[
            pl.BlockSpec((BN, D), lambda i: (i, 0)),
            pl.BlockSpec((8, D), lambda i: (0, 0)),
        ],
        out_shape=[
            jax.ShapeDtypeStruct((N, D), jnp.float32),
            jax.ShapeDtypeStruct((8, D), jnp.float32),
        ],
    )(aggp, sc, Wl2)


def _q2p_kernel(g_ref, hin_ref, sums_ref, bnw_ref, bnb_ref, wl1_ref, wsc_ref,
                h_ref, x1_ref, sc_ref, *, N):
    inv_n = 1.0 / N
    mean = sums_ref[0:1, :] * inv_n
    var = sums_ref[1:2, :] * inv_n - mean * mean
    rstd = lax.rsqrt(var + EPS)
    gn = (g_ref[...] - mean) * rstd * bnw_ref[...] + bnb_ref[...]
    h = gn + hin_ref[...]
    h_ref[...] = h
    x1_ref[...] = jnp.dot(h, wl1_ref[...], preferred_element_type=jnp.float32) * 0.25
    sc_ref[...] = jnp.dot(h, wsc_ref[...], preferred_element_type=jnp.float32) * (SIN8 * 0.25)


def _q2p(g, hin, sums, bnw, bnb, Wl1, Wsc, N, BN):
    return pl.pallas_call(
        functools.partial(_q2p_kernel, N=N),
        grid=(N // BN,),
        in_specs=[
            pl.BlockSpec((BN, D), lambda i: (i, 0)),
            pl.BlockSpec((BN, D), lambda i: (i, 0)),
            pl.BlockSpec((8, D), lambda i: (0, 0)),
            pl.BlockSpec((1, D), lambda i: (0, 0)),
            pl.BlockSpec((1, D), lambda i: (0, 0)),
            pl.BlockSpec((D, D), lambda i: (0, 0)),
            pl.BlockSpec((D, D), lambda i: (0, 0)),
        ],
        out_specs=[
            pl.BlockSpec((BN, D), lambda i: (i, 0)),
            pl.BlockSpec((BN, D), lambda i: (i, 0)),
            pl.BlockSpec((BN, D), lambda i: (i, 0)),
        ],
        out_shape=[
            jax.ShapeDtypeStruct((N, D), jnp.float32),
            jax.ShapeDtypeStruct((N, D), jnp.float32),
            jax.ShapeDtypeStruct((N, D), jnp.float32),
        ],
    )(g, hin, sums, bnw, bnb, Wl1, Wsc)


def _final_kernel(aggp_ref, sc_ref, wl2_ref, osum_ref, *, N):
    c = sc_ref[...] + jnp.dot(aggp_ref[...], wl2_ref[...],
                              preferred_element_type=jnp.float32) * (COS8 * 0.25)
    part = jnp.sum(c, axis=0) * (1.0 / (N ** 0.5))
    block = jnp.concatenate([part[None], jnp.zeros((7, D), jnp.float32)], axis=0)

    @pl.when(pl.program_id(0) == 0)
    def _():
        osum_ref[...] = jnp.zeros_like(osum_ref)

    osum_ref[...] += block


def _final(aggp, sc, Wl2, N, BN):
    return pl.pallas_call(
        functools.partial(_final_kernel, N=N),
        grid=(N // BN,),
        in_specs=[
            pl.BlockSpec((BN, D), lambda i: (i, 0)),
            pl.BlockSpec((BN, D), lambda i: (i, 0)),
            pl.BlockSpec((D, D), lambda i: (0, 0)),
        ],
        out_specs=pl.BlockSpec((8, D), lambda i: (0, 0)),
        out_shape=jax.ShapeDtypeStruct((8, D), jnp.float32),
    )(aggp, sc, Wl2)


def _sc_edge_call(N, EP):
    nchunks = EP // CHUNK
    NH = N // 2
    rpt8 = (NH // NS) & ~7
    tail_off = rpt8 * NS
    tail = NH - tail_off
    mesh = plsc.VectorSubcoreMesh(core_axis_name="c", subcore_axis_name="s")
    scale = (KTAB - 1) / S_MAX

    def body(x1_hbm, src_hbm, dst_hbm, s_hbm, tab_hbm, zero_hbm, out_hbm,
             src_v, dst_v, s_v, idx_v, tab_v, xg_v, ef_v,
             agg_sh, sem_a, sem_b):
        c = lax.axis_index("c")
        t = lax.axis_index("s")
        node_lo = c * NH
        pltpu.sync_copy(zero_hbm.at[pl.ds(t * rpt8, rpt8)],
                        agg_sh.at[pl.ds(t * rpt8, rpt8)])
        if tail:
            @pl.when(t == NS - 1)
            def _():
                pltpu.sync_copy(zero_hbm.at[pl.ds(tail_off, tail)],
                                agg_sh.at[pl.ds(tail_off, tail)])
        plsc.subcore_barrier()

        def chunk_body(ci, carry):
            b0 = t * EP + ci * CHUNK
            pltpu.sync_copy(src_hbm.at[pl.ds(b0, CHUNK)], src_v)
            pltpu.sync_copy(dst_hbm.at[pl.ds(b0, CHUNK)], dst_v)
            pltpu.sync_copy(s_hbm.at[pl.ds(b0, CHUNK)], s_v)

            def lane_body(k, carry2):
                sl = s_v[pl.ds(k * 16, 16)]
                p = sl * scale + 0.5
                i = jnp.minimum(p.astype(jnp.int32), KTAB - 1)
                loc = dst_v[pl.ds(k * 16, 16)] - node_lo
                ok = (loc >= 0) & (loc < NH)
                idx_v[pl.ds(k * 16, 16)] = jnp.where(ok, i, KTAB - 1)
                dst_v[pl.ds(k * 16, 16)] = jnp.where(ok, loc, 0)
                return carry2

            lax.fori_loop(0, CHUNK // 16, lane_body, 0, unroll=4)
            cp_a = pltpu.async_copy(tab_hbm.at[idx_v], tab_v, sem_a)
            cp_b = pltpu.async_copy(x1_hbm.at[src_v], xg_v, sem_b)
            cp_a.wait()
            cp_b.wait()

            def row_body(e, carry3):
                ef_v[e, :] = tab_v[e, :] * xg_v[e, :]
                return carry3

            lax.fori_loop(0, CHUNK, row_body, 0, unroll=8)
            pltpu.sync_copy(ef_v, agg_sh.at[dst_v], add=True)
            return carry

        lax.fori_loop(0, nchunks, chunk_body, 0)
        plsc.subcore_barrier()
        pltpu.sync_copy(agg_sh.at[pl.ds(t * rpt8, rpt8)],
                        out_hbm.at[pl.ds(node_lo + t * rpt8, rpt8)])
        if tail:
            @pl.when(t == NS - 1)
            def _():
                pltpu.sync_copy(agg_sh.at[pl.ds(tail_off, tail)],
                                out_hbm.at[pl.ds(node_lo + tail_off, tail)])

    return pl.kernel(
        body,
        out_type=jax.ShapeDtypeStruct((N, D), jnp.float32),
        mesh=mesh,
        compiler_params=pltpu.CompilerParams(use_tc_tiling_on_sc=False),
        scratch_types=[
            pltpu.VMEM((CHUNK,), jnp.int32),
            pltpu.VMEM((CHUNK,), jnp.int32),
            pltpu.VMEM((CHUNK,), jnp.float32),
            pltpu.VMEM((CHUNK,), jnp.int32),
            pltpu.VMEM((CHUNK, D), jnp.float32),
            pltpu.VMEM((CHUNK, D), jnp.float32),
            pltpu.VMEM((CHUNK, D), jnp.float32),
            pltpu.VMEM_SHARED((N // 2, D), jnp.float32),
            pltpu.SemaphoreType.DMA,
            pltpu.SemaphoreType.DMA,
        ],
    )


def kernel(x, pos, edge_index, edge_vec,
           W0_sc, W0_lin1, W0_fc1, W0_fc2, W0_lin2,
           W1_sc, W1_lin1, W1_fc1, W1_fc2, W1_lin2,
           W2_sc, W2_lin1, W2_fc1, W2_fc2, W2_lin2,
           bn0_w, bn0_b, bn1_w, bn1_b):
    N = x.shape[0]
    E = edge_index.shape[1]
    assert N % NS == 0
    BN = 5000
    assert N % BN == 0

    s = _edge_s(edge_vec, E)
    epad = -E % (NS * CHUNK)
    EPAD = E + epad
    src = edge_index[0]
    dst = edge_index[1]
    if epad:
        s = jnp.concatenate([s, jnp.full((epad,), 2.0 * S_MAX, jnp.float32)])
        src = jnp.concatenate([src, jnp.zeros((epad,), jnp.int32)])
        dst = jnp.concatenate([dst, jnp.zeros((epad,), jnp.int32)])
    EP = EPAD // NS

    tables = _build_tables(jnp.stack([W0_fc1, W1_fc1, W2_fc1]),
                           jnp.stack([W0_fc2, W1_fc2, W2_fc2]))
    zero_nd = jnp.zeros((N // 2, D), jnp.float32)
    sc_edge = _sc_edge_call(N, EP)

    x1, sc0 = _pre(x, W0_lin1, W0_sc, N, BN)
    aggp = sc_edge(x1, src, dst, s, tables[0], zero_nd)
    g0, sums0 = _q1(aggp, sc0, W0_lin2, N, BN)
    h1, x1, sc1 = _q2p(g0, x, sums0, bn0_w[None], bn0_b[None],
                       W1_lin1, W1_sc, N, BN)
    aggp = sc_edge(x1, src, dst, s, tables[1], zero_nd)
    g1, sums1 = _q1(aggp, sc1, W1_lin2, N, BN)
    _, x1, sc2 = _q2p(g1, h1, sums1, bn1_w[None], bn1_b[None],
                      W2_lin1, W2_sc, N, BN)
    aggp = sc_edge(x1, src, dst, s, tables[2], zero_nd)
    osum = _final(aggp, sc2, W2_lin2, N, BN)
    return osum[0:1, :]
